# baseline (speedup 1.0000x reference)
import functools

import jax
import jax.numpy as jnp
from jax import lax
from jax.experimental import pallas as pl
from jax.experimental.pallas import tpu as pltpu
from jax.experimental.pallas import tpu_sc as plsc

BATCH = 16384
N_FIELDS = 26
DIM = 16
LANES = 16
NROWS = 2600000

NUM_CORES = 2
NUM_SUBCORES = 16
NW = NUM_CORES * NUM_SUBCORES
BT = 128
NTILES = BATCH // BT
TILES_PER_W = NTILES // NW
FC = N_FIELDS * DIM


def _sc_gather(ids_t, tflat, off_b):
    mesh = plsc.VectorSubcoreMesh(core_axis_name="c", subcore_axis_name="s")

    @functools.partial(
        pl.kernel,
        mesh=mesh,
        out_type=jax.ShapeDtypeStruct(
            (N_FIELDS, DIM // 8, BT, 8, BT), jnp.float32
        ),
        scratch_types=[
            pltpu.VMEM((N_FIELDS, BT), jnp.int32),
            pltpu.VMEM((FC, BT), jnp.int32),
            pltpu.VMEM((N_FIELDS, DIM // 8, 8, BT), jnp.float32),
            pltpu.VMEM((N_FIELDS, LANES), jnp.int32),
            pltpu.VMEM((FC, LANES), jnp.int32),
            pltpu.SemaphoreType.DMA,
            pltpu.SemaphoreType.DMA,
        ],
        compiler_params=pltpu.CompilerParams(use_tc_tiling_on_sc=False),
    )
    def k(ids_hbm, tflat_hbm, off_hbm, out_hbm, idxf, idxg, obuf, off_v,
          bias, sem, osem):
        wid = lax.axis_index("s") * NUM_CORES + lax.axis_index("c")
        pltpu.sync_copy(off_hbm, off_v)

        def bias_body(fc, c0):
            f = lax.shift_right_logical(fc, 4)
            d = lax.bitwise_and(fc, 15)
            bias[fc, :] = off_v[f, :] + d * NROWS
            return c0

        lax.fori_loop(0, FC, bias_body, 0)

        def tile_body(t, carry):
            bt = wid * TILES_PER_W + t
            pltpu.sync_copy(ids_hbm.at[:, pl.ds(bt * BT, BT)], idxf)

            def addr_body(fc, c1):
                f = lax.shift_right_logical(fc, 4)
                for kk in range(BT // LANES):
                    sl = pl.ds(kk * LANES, LANES)
                    idxg[fc, sl] = idxf[f, sl] + bias[fc, :]
                return c1

            lax.fori_loop(0, FC, addr_body, 0)

            def fire_body(fc, c2):
                f = lax.shift_right_logical(fc, 4)
                d = lax.bitwise_and(fc, 15)
                dt = lax.shift_right_logical(d, 3)
                ds_ = lax.bitwise_and(d, 7)
                pltpu.async_copy(
                    tflat_hbm.at[idxg.at[fc]], obuf.at[f, dt, ds_], sem
                )
                return c2

            lax.fori_loop(0, FC, fire_body, 0)
            pltpu.make_async_copy(
                out_hbm.at[:, :, bt, :, :], obuf, sem
            ).wait()
            pltpu.sync_copy(obuf, out_hbm.at[:, :, bt, :, :])
            return carry

        lax.fori_loop(0, TILES_PER_W, tile_body, 0)

    return k(ids_t, tflat, off_b)


def kernel(hash_ids, table, offsets_buf):
    ids_t = hash_ids.T
    tflat = table.T.reshape(-1)
    off_b = jnp.broadcast_to(offsets_buf[:, None], (N_FIELDS, LANES))
    out5 = _sc_gather(ids_t, tflat, off_b)
    return jnp.transpose(out5, (2, 4, 0, 1, 3)).reshape(BATCH, N_FIELDS, DIM)


# speedup vs baseline: 2.7173x; 2.7173x over previous
import functools

import jax
import jax.numpy as jnp
from jax import lax
from jax.experimental import pallas as pl
from jax.experimental.pallas import tpu as pltpu
from jax.experimental.pallas import tpu_sc as plsc

BATCH = 16384
N_FIELDS = 26
DIM = 16
LANES = 16

NUM_CORES = 2
NUM_SUBCORES = 16
NW = NUM_CORES * NUM_SUBCORES
BT = 128
NTILES = BATCH // BT
TILES_PER_W = NTILES // NW


def _sc_gather(ids, table, off_b):
    mesh = plsc.VectorSubcoreMesh(core_axis_name="c", subcore_axis_name="s")

    @functools.partial(
        pl.kernel,
        mesh=mesh,
        out_type=jax.ShapeDtypeStruct(
            (N_FIELDS, DIM // 8, BT, 8, BT), jnp.float32
        ),
        scratch_types=[
            pltpu.VMEM((BT, N_FIELDS), jnp.int32),
            pltpu.VMEM((N_FIELDS, BT), jnp.int32),
            pltpu.VMEM((N_FIELDS, BT, DIM), jnp.float32),
            pltpu.VMEM((N_FIELDS, DIM // 8, 8, BT), jnp.float32),
            pltpu.VMEM((N_FIELDS, LANES), jnp.int32),
            pltpu.SemaphoreType.DMA,
        ],
        compiler_params=pltpu.CompilerParams(
            use_tc_tiling_on_sc=False, needs_layout_passes=False
        ),
    )
    def k(ids_hbm, table_hbm, off_hbm, out_hbm, idx2d, idxf, rows_v, obuf,
          off_v, sem):
        wid = lax.axis_index("s") * NUM_CORES + lax.axis_index("c")
        pltpu.sync_copy(off_hbm, off_v)
        iota = jax.lax.iota(jnp.int32, LANES)
        dt_idx = jax.lax.shift_right_logical(iota, 3)
        ds_idx = jax.lax.bitwise_and(iota, 7)

        def tile_body(t, carry):
            bt = wid * TILES_PER_W + t
            pltpu.sync_copy(ids_hbm.at[pl.ds(bt * BT, BT), :], idx2d)
            for f in range(N_FIELDS):
                fsplat = jnp.full((LANES,), f, dtype=jnp.int32)
                offf = off_v[f, :]
                for kk in range(BT // LANES):
                    bidx = iota + (kk * LANES)
                    g = plsc.load_gather(idx2d, [bidx, fsplat])
                    idxf[f, pl.ds(kk * LANES, LANES)] = g + offf
            copies = [
                pltpu.async_copy(table_hbm.at[idxf.at[f]], rows_v.at[f], sem)
                for f in range(N_FIELDS)
            ]
            for cp in copies:
                cp.wait()

            def b_body(b, c2):
                bs = jnp.full((LANES,), b, dtype=jnp.int32)
                for f in range(N_FIELDS):
                    fs = jnp.full((LANES,), f, dtype=jnp.int32)
                    row = rows_v[f, b, :]
                    plsc.store_scatter(obuf, [fs, dt_idx, ds_idx, bs], row)
                return c2

            lax.fori_loop(0, BT, b_body, 0)
            pltpu.sync_copy(obuf, out_hbm.at[:, :, bt, :, :])
            return carry

        lax.fori_loop(0, TILES_PER_W, tile_body, 0)

    return k(ids, table, off_b)


def kernel(hash_ids, table, offsets_buf):
    off_b = jnp.broadcast_to(offsets_buf[:, None], (N_FIELDS, LANES))
    out5 = _sc_gather(hash_ids, table, off_b)
    return jnp.transpose(out5, (2, 4, 0, 1, 3)).reshape(BATCH, N_FIELDS, DIM)
